# baseline (speedup 1.0000x reference)
import functools

import jax
import jax.numpy as jnp
from jax import lax
from jax.experimental import pallas as pl
from jax.experimental.pallas import tpu as pltpu
from jax.experimental.pallas import tpu_sc as plsc

NSEG = 27
NPAIR = 20000
UNITS = 128
ROWS = 512
COLS = 1024

NCORE = 2
NSUB = 16
SEG_PER_CORE = 14
PAIR_PER_SUB = NPAIR // NSUB
NCHUNK = 10
CHUNK = 125
CHUNK_PAD = 128
NLANE = 16
NVEC = UNITS // NLANE


def _sc_body(table_hbm, idx_hbm, out_hbm, idx_v, buf_v, acc_v, acc_sh, sem):
    c = lax.axis_index("c")
    s = lax.axis_index("s")

    zero16 = jnp.zeros((NLANE,), jnp.float32)

    def _zero(kk, carry):
        for u in range(NVEC):
            acc_v[kk, pl.ds(u * NLANE, NLANE)] = zero16
        return carry

    lax.fori_loop(0, NSUB, _zero, 0)

    @pl.when(s == 0)
    def _():
        pltpu.sync_copy(acc_v, acc_sh)

    plsc.subcore_barrier()

    def seg_body(kk, carry):
        k = c * SEG_PER_CORE + kk
        pltpu.sync_copy(idx_hbm.at[k, s], idx_v)

        def chunk_body(ch, acc):
            pltpu.async_copy(table_hbm.at[idx_v.at[ch]], buf_v, sem).wait()

            def row_body(r, a):
                return tuple(
                    a[u] + buf_v[r, pl.ds(u * NLANE, NLANE)]
                    for u in range(NVEC)
                )

            return lax.fori_loop(0, CHUNK, row_body, acc)

        acc = lax.fori_loop(0, NCHUNK, chunk_body,
                            tuple(zero16 for _ in range(NVEC)))
        for u in range(NVEC):
            acc_v[kk, pl.ds(u * NLANE, NLANE)] = acc[u]
        return carry

    lax.fori_loop(0, SEG_PER_CORE, seg_body, 0)

    row_ids = lax.iota(jnp.int32, NLANE)
    pltpu.sync_copy(acc_v, acc_sh.at[row_ids], add=True)
    plsc.subcore_barrier()

    @pl.when(s == 0)
    def _():
        pltpu.sync_copy(acc_sh, acc_v)
        inv = jnp.full((NLANE,), 1.0 / NPAIR, jnp.float32)

        def scale_body(kk, carry):
            for u in range(NVEC):
                sl = pl.ds(u * NLANE, NLANE)
                acc_v[kk, sl] = acc_v[kk, sl] * inv
            return carry

        lax.fori_loop(0, NSUB, scale_body, 0)
        pltpu.sync_copy(acc_v, out_hbm.at[c])


@jax.jit
def _node_pool_sc(table, idx4):
    mesh = plsc.VectorSubcoreMesh(core_axis_name="c", subcore_axis_name="s")
    k = functools.partial(
        pl.kernel,
        out_type=jax.ShapeDtypeStruct((NCORE, NSUB, UNITS), jnp.float32),
        mesh=mesh,
        scratch_types=[
            pltpu.VMEM((NCHUNK, CHUNK_PAD), jnp.int32),
            pltpu.VMEM((CHUNK_PAD, UNITS), jnp.float32),
            pltpu.VMEM((NSUB, UNITS), jnp.float32),
            pltpu.VMEM_SHARED((NSUB, UNITS), jnp.float32),
            pltpu.SemaphoreType.DMA,
        ],
    )(_sc_body)
    return k(table, idx4)


def kernel(inputs, pairs):
    table = inputs.reshape(ROWS * COLS, UNITS)
    flat = pairs[..., 0] * COLS + pairs[..., 1]
    flat = jnp.concatenate(
        [flat, jnp.zeros((1, NPAIR), jnp.int32)], axis=0)
    idx4 = flat.reshape(NCORE * SEG_PER_CORE, NSUB, NCHUNK, CHUNK)
    idx4 = jnp.pad(idx4, ((0, 0), (0, 0), (0, 0), (0, CHUNK_PAD - CHUNK)))
    out = _node_pool_sc(table, idx4)
    return out[:, :SEG_PER_CORE].reshape(NCORE * SEG_PER_CORE, UNITS)[:NSEG]
